# Initial kernel scaffold; baseline (speedup 1.0000x reference)
import jax
import jax.numpy as jnp
from jax import lax
from jax.experimental import pallas as pl
from jax.experimental.pallas import tpu as pltpu
from jax.experimental.pallas import tpu_sc as plsc

N = 10000
E = 320000
D = 128
D_OUT = 40

NC = 2
NS = 16
NW = NC * NS
EPW = E // NW
IB = 80
NCHUNK = EPW // IB
RPT = N // NS

ROW_BLOCK = 400
GRID = N // ROW_BLOCK


def _scatter_body(compute_deg, p_hbm, src_hbm, dst_hbm, zrow_hbm, ones_hbm,
                  out_hbm, deg_hbm, srcv, dstv, rows, onesv, acc, dacc, gsem):
    cid = lax.axis_index("c")
    sid = lax.axis_index("s")
    wid = cid * NS + sid
    r0 = sid * RPT

    pltpu.sync_copy(zrow_hbm.at[pl.ds(r0, RPT)], acc.at[pl.ds(r0, RPT)])
    if compute_deg:
        pltpu.sync_copy(zrow_hbm.at[pl.ds(r0, RPT), pl.ds(0, 16)],
                        dacc.at[pl.ds(r0, RPT)])
        pltpu.sync_copy(ones_hbm, onesv)
    pltpu.sync_copy(src_hbm.at[wid], srcv)
    pltpu.sync_copy(dst_hbm.at[wid], dstv)
    plsc.subcore_barrier()

    pltpu.async_copy(p_hbm.at[srcv.at[0]], rows.at[0], gsem)

    def step(j, carry):
        par = lax.rem(j, 2)
        nxt = lax.rem(j + 1, 2)

        @pl.when(j + 1 < NCHUNK)
        def _():
            pltpu.async_copy(p_hbm.at[srcv.at[j + 1]], rows.at[nxt], gsem)

        pltpu.make_async_copy(p_hbm.at[srcv.at[j]], rows.at[par], gsem).wait()
        pltpu.sync_copy(rows.at[par], acc.at[dstv.at[j]], add=True)
        if compute_deg:
            pltpu.sync_copy(onesv, dacc.at[dstv.at[j]], add=True)
        return carry

    lax.fori_loop(0, NCHUNK, step, 0)
    plsc.subcore_barrier()

    pltpu.sync_copy(acc.at[pl.ds(r0, RPT)], out_hbm.at[cid, pl.ds(r0, RPT)])
    if compute_deg:
        pltpu.sync_copy(dacc.at[pl.ds(r0, RPT)], deg_hbm.at[cid, pl.ds(r0, RPT)])


def _make_scatter(compute_deg):
    mesh = plsc.VectorSubcoreMesh(core_axis_name="c", subcore_axis_name="s",
                                  num_cores=NC, num_subcores=NS)
    out_type = [jax.ShapeDtypeStruct((NC, N, D), jnp.float32)]
    if compute_deg:
        out_type.append(jax.ShapeDtypeStruct((NC, N, 16), jnp.float32))
    scratch = [
        pltpu.VMEM((NCHUNK, IB), jnp.int32),
        pltpu.VMEM((NCHUNK, IB), jnp.int32),
        pltpu.VMEM((2, IB, D), jnp.float32),
        pltpu.VMEM((IB, 16), jnp.float32),
        pltpu.VMEM_SHARED((N, D), jnp.float32),
        pltpu.VMEM_SHARED((N, 16), jnp.float32),
        pltpu.SemaphoreType.DMA,
    ]

    if compute_deg:
        def body(p, src, dst, zrow, ones, out, deg, *rest):
            _scatter_body(True, p, src, dst, zrow, ones, out, deg, *rest)
    else:
        def body(p, src, dst, zrow, ones, out, *rest):
            _scatter_body(False, p, src, dst, zrow, ones, out, None, *rest)

    return pl.kernel(body, out_type=out_type, mesh=mesh, scratch_types=scratch)


_scatter_deg = _make_scatter(True)
_scatter_nodeg = _make_scatter(False)


def _mm2_body(x_ref, wl_ref, wr_ref, p_ref, q_ref):
    xb = x_ref[...]
    p_ref[...] = jnp.dot(xb, wl_ref[...], preferred_element_type=jnp.float32)
    q_ref[...] = jnp.dot(xb, wr_ref[...], preferred_element_type=jnp.float32)


def _mm2(x, wl, wr):
    return pl.pallas_call(
        _mm2_body,
        grid=(GRID,),
        in_specs=[
            pl.BlockSpec((ROW_BLOCK, D), lambda i: (i, 0)),
            pl.BlockSpec((D, D), lambda i: (0, 0)),
            pl.BlockSpec((D, D), lambda i: (0, 0)),
        ],
        out_specs=[
            pl.BlockSpec((ROW_BLOCK, D), lambda i: (i, 0)),
            pl.BlockSpec((ROW_BLOCK, D), lambda i: (i, 0)),
        ],
        out_shape=[
            jax.ShapeDtypeStruct((N, D), jnp.float32),
            jax.ShapeDtypeStruct((N, D), jnp.float32),
        ],
    )(x, wl, wr)


def _mid_body(aggp_ref, degp_ref, q_ref, bl_ref, wl1_ref, wr1_ref, fct_ref,
              p1_ref, q1_ref, op_ref, deg_ref):
    agg = aggp_ref[0] + aggp_ref[1]
    deg16 = degp_ref[0] + degp_ref[1]
    deg = jnp.maximum(deg16[:, :1], 1.0)
    h0 = jnp.maximum(agg / deg + q_ref[...] + bl_ref[...], 0.0)
    p1_ref[...] = jnp.dot(h0, wl1_ref[...], preferred_element_type=jnp.float32)
    q1_ref[...] = jnp.dot(h0, wr1_ref[...], preferred_element_type=jnp.float32)
    op_ref[...] = jnp.dot(h0, fct_ref[...], preferred_element_type=jnp.float32)
    deg_ref[...] = deg16


def _mid(aggp, degp, q0, bl0, wl1, wr1, fc_top):
    return pl.pallas_call(
        _mid_body,
        grid=(GRID,),
        in_specs=[
            pl.BlockSpec((NC, ROW_BLOCK, D), lambda i: (0, i, 0)),
            pl.BlockSpec((NC, ROW_BLOCK, 16), lambda i: (0, i, 0)),
            pl.BlockSpec((ROW_BLOCK, D), lambda i: (i, 0)),
            pl.BlockSpec((1, D), lambda i: (0, 0)),
            pl.BlockSpec((D, D), lambda i: (0, 0)),
            pl.BlockSpec((D, D), lambda i: (0, 0)),
            pl.BlockSpec((D, D_OUT), lambda i: (0, 0)),
        ],
        out_specs=[
            pl.BlockSpec((ROW_BLOCK, D), lambda i: (i, 0)),
            pl.BlockSpec((ROW_BLOCK, D), lambda i: (i, 0)),
            pl.BlockSpec((ROW_BLOCK, D_OUT), lambda i: (i, 0)),
            pl.BlockSpec((ROW_BLOCK, 16), lambda i: (i, 0)),
        ],
        out_shape=[
            jax.ShapeDtypeStruct((N, D), jnp.float32),
            jax.ShapeDtypeStruct((N, D), jnp.float32),
            jax.ShapeDtypeStruct((N, D_OUT), jnp.float32),
            jax.ShapeDtypeStruct((N, 16), jnp.float32),
        ],
    )(aggp, degp, q0, bl0, wl1, wr1, fc_top)


def _out_body(aggp_ref, deg_ref, q_ref, bl_ref, op_ref, fcb_ref, b_ref, o_ref):
    agg = aggp_ref[0] + aggp_ref[1]
    deg = jnp.maximum(deg_ref[:, :1], 1.0)
    h1 = jnp.maximum(agg / deg + q_ref[...] + bl_ref[...], 0.0)
    o_ref[...] = (op_ref[...] + b_ref[...]
                  + jnp.dot(h1, fcb_ref[...], preferred_element_type=jnp.float32))


def _out(aggp, deg16, q1, bl1, opart, fc_bot, fc1_b):
    return pl.pallas_call(
        _out_body,
        grid=(GRID,),
        in_specs=[
            pl.BlockSpec((NC, ROW_BLOCK, D), lambda i: (0, i, 0)),
            pl.BlockSpec((ROW_BLOCK, 16), lambda i: (i, 0)),
            pl.BlockSpec((ROW_BLOCK, D), lambda i: (i, 0)),
            pl.BlockSpec((1, D), lambda i: (0, 0)),
            pl.BlockSpec((ROW_BLOCK, D_OUT), lambda i: (i, 0)),
            pl.BlockSpec((D, D_OUT), lambda i: (0, 0)),
            pl.BlockSpec((1, D_OUT), lambda i: (0, 0)),
        ],
        out_specs=pl.BlockSpec((ROW_BLOCK, D_OUT), lambda i: (i, 0)),
        out_shape=jax.ShapeDtypeStruct((N, D_OUT), jnp.float32),
    )(aggp, deg16, q1, bl1, opart, fc_bot, fc1_b)


@jax.jit
def kernel(x, edge_index, wl0, bl0, wr0, wl1, bl1, wr1, fc1_w, fc1_b):
    src = edge_index[0].reshape(NW, NCHUNK, IB)
    dst = edge_index[1].reshape(NW, NCHUNK, IB)
    zrow = jnp.zeros((N, D), jnp.float32)
    ones = jnp.ones((IB, 16), jnp.float32)

    p0, q0 = _mm2(x, wl0, wr0)
    agg0p, degp = _scatter_deg(p0, src, dst, zrow, ones)
    p1, q1, opart, deg16 = _mid(agg0p, degp, q0, bl0.reshape(1, D), wl1, wr1,
                                fc1_w[:D])
    agg1p = _scatter_nodeg(p1, src, dst, zrow, ones)
    return _out(agg1p, deg16, q1, bl1.reshape(1, D), opart, fc1_w[D:],
                fc1_b.reshape(1, D_OUT))


# speedup vs baseline: 3.9148x; 3.9148x over previous
import jax
import jax.numpy as jnp
from jax import lax
from jax.experimental import pallas as pl
from jax.experimental.pallas import tpu as pltpu
from jax.experimental.pallas import tpu_sc as plsc

N = 10000
E = 320000
D = 128
D_OUT = 40

NC = 2
NS = 16
NW = NC * NS
IB = 128
NCHUNK = 80
EPAD = NW * NCHUNK * IB
NPAD = 10240
RPT = NPAD // NS

ROW_BLOCK = 400
GRID = N // ROW_BLOCK

_MESH = dict(core_axis_name="c", subcore_axis_name="s",
             num_cores=NC, num_subcores=NS)


def _agg_body(p_hbm, e_hbm, zrow_hbm, out_hbm, idxa, idxb, rowsa, rowsb,
              acc, gsem, isem):
    cid = lax.axis_index("c")
    sid = lax.axis_index("s")
    wid = cid * NS + sid
    r0 = sid * RPT

    pltpu.sync_copy(zrow_hbm.at[pl.ds(r0, RPT)], acc.at[pl.ds(r0, RPT)])

    pltpu.sync_copy(e_hbm.at[wid, 0], idxa)
    pltpu.async_copy(p_hbm.at[idxa.at[0]], rowsa, gsem)
    pltpu.async_copy(e_hbm.at[wid, 1], idxb, isem)
    plsc.subcore_barrier()

    def substep(c, idx_c, rows_c, idx_n, rows_n):
        @pl.when(c + 1 < NCHUNK)
        def _():
            pltpu.make_async_copy(e_hbm.at[wid, c + 1], idx_n, isem).wait()
            pltpu.async_copy(p_hbm.at[idx_n.at[0]], rows_n, gsem)

        pltpu.make_async_copy(p_hbm.at[idx_c.at[0]], rows_c, gsem).wait()
        pltpu.sync_copy(rows_c, acc.at[idx_c.at[1]], add=True)

        @pl.when(c + 2 < NCHUNK)
        def _():
            pltpu.async_copy(e_hbm.at[wid, c + 2], idx_c, isem)

    def step(g, carry):
        substep(2 * g, idxa, rowsa, idxb, rowsb)
        substep(2 * g + 1, idxb, rowsb, idxa, rowsa)
        return carry

    lax.fori_loop(0, NCHUNK // 2, step, 0)
    plsc.subcore_barrier()

    pltpu.sync_copy(acc.at[pl.ds(r0, RPT)], out_hbm.at[cid, pl.ds(r0, RPT)])


def _make_agg():
    return pl.kernel(
        _agg_body,
        out_type=jax.ShapeDtypeStruct((NC, NPAD, D), jnp.float32),
        mesh=plsc.VectorSubcoreMesh(**_MESH),
        scratch_types=[
            pltpu.VMEM((2, IB), jnp.int32),
            pltpu.VMEM((2, IB), jnp.int32),
            pltpu.VMEM((IB, D), jnp.float32),
            pltpu.VMEM((IB, D), jnp.float32),
            pltpu.VMEM_SHARED((NPAD, D), jnp.float32),
            pltpu.SemaphoreType.DMA,
            pltpu.SemaphoreType.DMA,
        ],
    )


def _deg_body(e_hbm, zrow_hbm, ones_hbm, out_hbm, idxa, idxb, onesv, acc,
              isem):
    cid = lax.axis_index("c")
    sid = lax.axis_index("s")
    wid = cid * NS + sid
    r0 = sid * RPT

    pltpu.sync_copy(zrow_hbm.at[pl.ds(r0, RPT)], acc.at[pl.ds(r0, RPT)])
    pltpu.sync_copy(ones_hbm, onesv)

    pltpu.sync_copy(e_hbm.at[wid, 0, 1], idxa)
    pltpu.async_copy(e_hbm.at[wid, 1, 1], idxb, isem)
    plsc.subcore_barrier()

    def substep(c, idx_c, idx_n):
        @pl.when(c + 1 < NCHUNK)
        def _():
            pltpu.make_async_copy(e_hbm.at[wid, c + 1, 1], idx_n, isem).wait()

        pltpu.sync_copy(onesv, acc.at[idx_c], add=True)

        @pl.when(c + 2 < NCHUNK)
        def _():
            pltpu.async_copy(e_hbm.at[wid, c + 2, 1], idx_c, isem)

    def step(g, carry):
        substep(2 * g, idxa, idxb)
        substep(2 * g + 1, idxb, idxa)
        return carry

    lax.fori_loop(0, NCHUNK // 2, step, 0)
    plsc.subcore_barrier()

    pltpu.sync_copy(acc.at[pl.ds(r0, RPT)], out_hbm.at[cid, pl.ds(r0, RPT)])


def _make_deg():
    return pl.kernel(
        _deg_body,
        out_type=jax.ShapeDtypeStruct((NC, NPAD, D), jnp.float32),
        mesh=plsc.VectorSubcoreMesh(**_MESH),
        scratch_types=[
            pltpu.VMEM((IB,), jnp.int32),
            pltpu.VMEM((IB,), jnp.int32),
            pltpu.VMEM((IB, D), jnp.float32),
            pltpu.VMEM_SHARED((NPAD, D), jnp.float32),
            pltpu.SemaphoreType.DMA,
        ],
    )


_agg = _make_agg()
_deg = _make_deg()


def _mm2_body(x_ref, wl_ref, wr_ref, p_ref, q_ref):
    xb = x_ref[...]
    p_ref[...] = jnp.dot(xb, wl_ref[...], preferred_element_type=jnp.float32)
    q_ref[...] = jnp.dot(xb, wr_ref[...], preferred_element_type=jnp.float32)


def _mm2(x, wl, wr):
    return pl.pallas_call(
        _mm2_body,
        grid=(GRID,),
        in_specs=[
            pl.BlockSpec((ROW_BLOCK, D), lambda i: (i, 0)),
            pl.BlockSpec((D, D), lambda i: (0, 0)),
            pl.BlockSpec((D, D), lambda i: (0, 0)),
        ],
        out_specs=[
            pl.BlockSpec((ROW_BLOCK, D), lambda i: (i, 0)),
            pl.BlockSpec((ROW_BLOCK, D), lambda i: (i, 0)),
        ],
        out_shape=[
            jax.ShapeDtypeStruct((N, D), jnp.float32),
            jax.ShapeDtypeStruct((N, D), jnp.float32),
        ],
    )(x, wl, wr)


def _mid_body(aggp_ref, degp_ref, q_ref, bl_ref, wl1_ref, wr1_ref, fct_ref,
              p1_ref, q1_ref, op_ref):
    agg = aggp_ref[0] + aggp_ref[1]
    deg = jnp.maximum(degp_ref[0] + degp_ref[1], 1.0)
    h0 = jnp.maximum(agg / deg + q_ref[...] + bl_ref[...], 0.0)
    p1_ref[...] = jnp.dot(h0, wl1_ref[...], preferred_element_type=jnp.float32)
    q1_ref[...] = jnp.dot(h0, wr1_ref[...], preferred_element_type=jnp.float32)
    op_ref[...] = jnp.dot(h0, fct_ref[...], preferred_element_type=jnp.float32)


def _mid(aggp, degp, q0, bl0, wl1, wr1, fc_top):
    return pl.pallas_call(
        _mid_body,
        grid=(GRID,),
        in_specs=[
            pl.BlockSpec((NC, ROW_BLOCK, D), lambda i: (0, i, 0)),
            pl.BlockSpec((NC, ROW_BLOCK, D), lambda i: (0, i, 0)),
            pl.BlockSpec((ROW_BLOCK, D), lambda i: (i, 0)),
            pl.BlockSpec((1, D), lambda i: (0, 0)),
            pl.BlockSpec((D, D), lambda i: (0, 0)),
            pl.BlockSpec((D, D), lambda i: (0, 0)),
            pl.BlockSpec((D, D_OUT), lambda i: (0, 0)),
        ],
        out_specs=[
            pl.BlockSpec((ROW_BLOCK, D), lambda i: (i, 0)),
            pl.BlockSpec((ROW_BLOCK, D), lambda i: (i, 0)),
            pl.BlockSpec((ROW_BLOCK, D_OUT), lambda i: (i, 0)),
        ],
        out_shape=[
            jax.ShapeDtypeStruct((N, D), jnp.float32),
            jax.ShapeDtypeStruct((N, D), jnp.float32),
            jax.ShapeDtypeStruct((N, D_OUT), jnp.float32),
        ],
    )(aggp, degp, q0, bl0, wl1, wr1, fc_top)


def _out_body(aggp_ref, degp_ref, q_ref, bl_ref, op_ref, fcb_ref, b_ref,
              o_ref):
    agg = aggp_ref[0] + aggp_ref[1]
    deg = jnp.maximum(degp_ref[0] + degp_ref[1], 1.0)
    h1 = jnp.maximum(agg / deg + q_ref[...] + bl_ref[...], 0.0)
    o_ref[...] = (op_ref[...] + b_ref[...]
                  + jnp.dot(h1, fcb_ref[...], preferred_element_type=jnp.float32))


def _out(aggp, degp, q1, bl1, opart, fc_bot, fc1_b):
    return pl.pallas_call(
        _out_body,
        grid=(GRID,),
        in_specs=[
            pl.BlockSpec((NC, ROW_BLOCK, D), lambda i: (0, i, 0)),
            pl.BlockSpec((NC, ROW_BLOCK, D), lambda i: (0, i, 0)),
            pl.BlockSpec((ROW_BLOCK, D), lambda i: (i, 0)),
            pl.BlockSpec((1, D), lambda i: (0, 0)),
            pl.BlockSpec((ROW_BLOCK, D_OUT), lambda i: (i, 0)),
            pl.BlockSpec((D, D_OUT), lambda i: (0, 0)),
            pl.BlockSpec((1, D_OUT), lambda i: (0, 0)),
        ],
        out_specs=pl.BlockSpec((ROW_BLOCK, D_OUT), lambda i: (i, 0)),
        out_shape=jax.ShapeDtypeStruct((N, D_OUT), jnp.float32),
    )(aggp, degp, q1, bl1, opart, fc_bot, fc1_b)


@jax.jit
def kernel(x, edge_index, wl0, bl0, wr0, wl1, bl1, wr1, fc1_w, fc1_b):
    npad_e = EPAD - E
    pad_src = jnp.zeros((npad_e,), edge_index.dtype)
    pad_dst = N + (jnp.arange(npad_e, dtype=edge_index.dtype) % 16)
    src_p = jnp.concatenate([edge_index[0], pad_src])
    dst_p = jnp.concatenate([edge_index[1], pad_dst])
    edges = (jnp.stack([src_p, dst_p])
             .reshape(2, NW, NCHUNK, IB).transpose(1, 2, 0, 3))
    zrow = jnp.zeros((NPAD, D), jnp.float32)
    ones = jnp.ones((IB, D), jnp.float32)

    degp = _deg(edges, zrow, ones)
    p0, q0 = _mm2(x, wl0, wr0)
    agg0p = _agg(p0, edges, zrow)
    p1, q1, opart = _mid(agg0p, degp, q0, bl0.reshape(1, D), wl1, wr1,
                         fc1_w[:D])
    agg1p = _agg(p1, edges, zrow)
    return _out(agg1p, degp, q1, bl1.reshape(1, D), opart, fc1_w[D:],
                fc1_b.reshape(1, D_OUT))
